# Initial kernel scaffold; baseline (speedup 1.0000x reference)
import jax
import jax.numpy as jnp
from jax.experimental import pallas as pl


def kernel(X_gene_batch, layer_weight, node_bias, head_W, head_b, gene_map, srcs, dst_uniques, dst_poss, root_ids):
    raise NotImplementedError("write your pallas kernel here")


# speedup vs baseline: 2.0706x; 2.0706x over previous
import functools

import jax
import jax.numpy as jnp
from jax import lax
from jax.experimental import pallas as pl
from jax.experimental.pallas import tpu as pltpu
from jax.experimental.pallas import tpu_sc as plsc

B, G, N, L, E, U, R, C = 64, 20000, 50000, 8, 100000, 25000, 128, 2

NLANE = 16
NCORE = 2
NSUB = 16
NPASS = (B // NLANE) // NCORE

E_PAD = 102400
EC, ECW = 50, 128
U_PAD = 25088
UC, UCW = 14, 112
UPT = UC * UCW
G_PAD = 20480
GC = 10
N_PAD = 50176
ZROWS = 784
H_GARB = N
A_GARB = U


def _splat(i):
    return jnp.full((NLANE,), i, jnp.int32)


def _tanh16(x):
    a = jnp.abs(x)
    e = jnp.exp(-2.0 * a)
    t = (1.0 - e) / (1.0 + e)
    return jnp.where(x < 0, -t, t)


def _sc_body(x4, lw16, wf, hb16, gidx, sidx, pidx, duidx, biasl, ridx, zro,
             out, h_sh, agg_sh, idx_s, idx_p, rows, zbuf, idx_du, bias_v,
             ubuf, lw_v, wf_v, hb_v, ridx_v, ostage):
    cid = lax.axis_index("c")
    sid = lax.axis_index("s")

    pltpu.sync_copy(lw16, lw_v)
    pltpu.sync_copy(wf, wf_v)
    pltpu.sync_copy(hb16, hb_v)
    pltpu.sync_copy(zro, zbuf)
    pltpu.sync_copy(ridx, ridx_v)

    def one_pass(p, _):
        k = 2 * p + cid

        def zrow(t, _):
            pltpu.sync_copy(zbuf, h_sh.at[pl.ds(sid * 4 * ZROWS + t * ZROWS,
                                                ZROWS)])
            return ()
        lax.fori_loop(0, 4, zrow, ())
        plsc.subcore_barrier()

        pltpu.sync_copy(gidx.at[sid], idx_s.at[pl.ds(0, GC)])

        def gscat(j, _):
            pltpu.sync_copy(x4.at[k, pl.ds(sid * GC * ECW + j * ECW, ECW)],
                            rows)
            pltpu.sync_copy(rows, h_sh.at[idx_s.at[j]])
            return ()
        lax.fori_loop(0, GC, gscat, ())
        plsc.subcore_barrier()

        def layer(li, _):
            pltpu.sync_copy(zbuf, agg_sh.at[pl.ds(sid * UPT, ZROWS)])
            pltpu.sync_copy(zbuf, agg_sh.at[pl.ds(sid * UPT + ZROWS, ZROWS)])
            plsc.subcore_barrier()

            pltpu.sync_copy(sidx.at[li, sid], idx_s)
            pltpu.sync_copy(pidx.at[li, sid], idx_p)

            def edge(j, _):
                pltpu.sync_copy(h_sh.at[idx_s.at[j]], rows)
                pltpu.sync_copy(rows, agg_sh.at[idx_p.at[j]], add=True)
                return ()
            lax.fori_loop(0, EC, edge, ())
            plsc.subcore_barrier()

            pltpu.sync_copy(duidx.at[li, sid], idx_du)
            pltpu.sync_copy(biasl.at[li, sid], bias_v)
            w_spl = plsc.load_gather(lw_v, [_splat(li)])

            def uchunk(cc, _):
                pltpu.sync_copy(agg_sh.at[pl.ds(sid * UPT + cc * UCW, UCW)],
                                ubuf)

                def urow(i, _):
                    v = ubuf[i, :]
                    bs = plsc.load_gather(bias_v, [_splat(cc * UCW + i)])
                    ubuf[i, :] = _tanh16(v * w_spl + bs)
                    return ()
                lax.fori_loop(0, UCW, urow, ())
                pltpu.sync_copy(ubuf, h_sh.at[idx_du.at[cc]])
                return ()
            lax.fori_loop(0, UC, uchunk, ())
            plsc.subcore_barrier()
            return ()
        lax.fori_loop(0, L, layer, ())

        @pl.when(sid == 0)
        def _():
            pltpu.sync_copy(h_sh.at[ridx_v], rows)

            def mm(r, acc):
                a0, a1 = acc
                v = rows[r, :]
                w0 = plsc.load_gather(wf_v, [_splat(2 * r)])
                w1 = plsc.load_gather(wf_v, [_splat(2 * r + 1)])
                return (a0 + v * w0, a1 + v * w1)
            a0, a1 = lax.fori_loop(
                0, R, mm, (jnp.zeros((NLANE,), jnp.float32),
                           jnp.zeros((NLANE,), jnp.float32)))
            b0 = plsc.load_gather(hb_v, [_splat(0)])
            b1 = plsc.load_gather(hb_v, [_splat(1)])
            ostage[...] = a0 + b0
            pltpu.sync_copy(ostage, out.at[k, 0])
            ostage[...] = a1 + b1
            pltpu.sync_copy(ostage, out.at[k, 1])
        plsc.subcore_barrier()
        return ()
    lax.fori_loop(0, NPASS, one_pass, ())


_sc_call = functools.partial(
    pl.kernel,
    out_type=jax.ShapeDtypeStruct((NPASS * NCORE, C, NLANE), jnp.float32),
    mesh=plsc.VectorSubcoreMesh(core_axis_name="c", subcore_axis_name="s"),
    compiler_params=pltpu.CompilerParams(needs_layout_passes=False,
                                         use_tc_tiling_on_sc=False),
    scratch_types=[
        pltpu.VMEM_SHARED((N_PAD, NLANE), jnp.float32),
        pltpu.VMEM_SHARED((U_PAD, NLANE), jnp.float32),
        pltpu.VMEM((EC, ECW), jnp.int32),
        pltpu.VMEM((EC, ECW), jnp.int32),
        pltpu.VMEM((ECW, NLANE), jnp.float32),
        pltpu.VMEM((ZROWS, NLANE), jnp.float32),
        pltpu.VMEM((UC, UCW), jnp.int32),
        pltpu.VMEM((UPT,), jnp.float32),
        pltpu.VMEM((UCW, NLANE), jnp.float32),
        pltpu.VMEM((NLANE,), jnp.float32),
        pltpu.VMEM((2 * R,), jnp.float32),
        pltpu.VMEM((NLANE,), jnp.float32),
        pltpu.VMEM((R,), jnp.int32),
        pltpu.VMEM((NLANE,), jnp.float32),
    ],
)(_sc_body)


def _pad1(a, n, val):
    return jnp.concatenate(
        [a, jnp.full((n - a.shape[0],), val, a.dtype)])


def kernel(X_gene_batch, layer_weight, node_bias, head_W, head_b,
           gene_map, srcs, dst_uniques, dst_poss, root_ids):
    f32, i32 = jnp.float32, jnp.int32

    xt = jnp.zeros((G_PAD, B), f32).at[:G].set(X_gene_batch.T)
    x4 = xt.reshape(G_PAD, NPASS * NCORE, NLANE).transpose(1, 0, 2)

    gidx = _pad1(gene_map.astype(i32), G_PAD, H_GARB).reshape(NSUB, GC, ECW)

    epad = jnp.full((L, E_PAD - E), H_GARB, i32)
    sidx = jnp.concatenate([srcs.astype(i32), epad], axis=1)
    sidx = sidx.reshape(L, NSUB, EC, ECW)
    ppad = jnp.full((L, E_PAD - E), A_GARB, i32)
    pidx = jnp.concatenate([dst_poss.astype(i32), ppad], axis=1)
    pidx = pidx.reshape(L, NSUB, EC, ECW)

    upad = jnp.full((L, U_PAD - U), H_GARB, i32)
    duidx = jnp.concatenate([dst_uniques.astype(i32), upad], axis=1)
    biasl = jnp.take(node_bias.astype(f32), duidx, axis=0,
                     mode="fill", fill_value=0.0)
    biasl = biasl.reshape(L, NSUB, UPT)
    duidx = duidx.reshape(L, NSUB, UC, UCW)

    lw16 = _pad1(layer_weight.astype(f32), NLANE, 0.0)
    wf = head_W.astype(f32).reshape(2 * R)
    hb16 = _pad1(head_b.astype(f32), NLANE, 0.0)
    ridx = root_ids.astype(i32)
    zro = jnp.zeros((ZROWS, NLANE), f32)

    out4 = _sc_call(x4, lw16, wf, hb16, gidx, sidx, pidx, duidx, biasl,
                    ridx, zro)
    return out4.transpose(0, 2, 1).reshape(B, C)


# speedup vs baseline: 4.4250x; 2.1371x over previous
import functools

import jax
import jax.numpy as jnp
from jax import lax
from jax.experimental import pallas as pl
from jax.experimental.pallas import tpu as pltpu
from jax.experimental.pallas import tpu_sc as plsc

B, G, N, L, E, U, R, C = 64, 20000, 50000, 8, 100000, 25000, 128, 2

NLANE = 16
NCORE = 2
NSUB = 16
NPASS = (B // NLANE) // NCORE

E_PAD = 102400
EC, ECW = 50, 128
U_PAD = 25088
UC, UCW = 14, 112
UPT = UC * UCW
G_PAD = 20480
GC = 10
N_PAD = 50176
ZROWS = 784
H_GARB = N
A_GARB = U


def _splat(i):
    return jnp.full((NLANE,), i, jnp.int32)


def _tanh16(x):
    a = jnp.abs(x)
    e = jnp.exp(-2.0 * a)
    t = (1.0 - e) / (1.0 + e)
    return jnp.where(x < 0, -t, t)


def _sc_body(x4, lw16, wf, hb16, gidx, sidx, pidx, duidx, nb, ridx, zro,
             out, h_sh, agg_sh, idx_s, idx_p, rows, rows_b, zbuf, idx_du,
             bias_v, ubuf, lw_v, wf_v, hb_v, ridx_v, ostage,
             gsem_a, gsem_b, ssem_a, ssem_b):
    cid = lax.axis_index("c")
    sid = lax.axis_index("s")

    pltpu.sync_copy(lw16, lw_v)
    pltpu.sync_copy(wf, wf_v)
    pltpu.sync_copy(hb16, hb_v)
    pltpu.sync_copy(zro, zbuf)
    pltpu.sync_copy(ridx, ridx_v)

    def one_pass(p, _):
        k = 2 * p + cid

        def zrow(t, _):
            pltpu.sync_copy(zbuf, h_sh.at[pl.ds(sid * 4 * ZROWS + t * ZROWS,
                                                ZROWS)])
            return ()
        lax.fori_loop(0, 4, zrow, ())
        plsc.subcore_barrier()

        pltpu.sync_copy(gidx.at[sid], idx_s.at[pl.ds(0, GC)])

        def gscat(j, _):
            pltpu.sync_copy(x4.at[k, pl.ds(sid * GC * ECW + j * ECW, ECW)],
                            rows)
            pltpu.sync_copy(rows, h_sh.at[idx_s.at[j]])
            return ()
        lax.fori_loop(0, GC, gscat, ())
        plsc.subcore_barrier()

        def layer(li, _):
            pltpu.sync_copy(zbuf, agg_sh.at[pl.ds(sid * UPT, ZROWS)])
            pltpu.sync_copy(zbuf, agg_sh.at[pl.ds(sid * UPT + ZROWS, ZROWS)])
            plsc.subcore_barrier()

            pltpu.sync_copy(sidx.at[li, sid], idx_s)
            pltpu.sync_copy(pidx.at[li, sid], idx_p)

            def _g(j, buf, sem):
                return pltpu.async_copy(h_sh.at[idx_s.at[j]], buf, sem)

            def _s(buf, j, sem):
                return pltpu.async_copy(buf, agg_sh.at[idx_p.at[j]], sem,
                                        add=True)

            _g(0, rows, gsem_a)

            def edge2(t, _):
                a = 2 * t
                b = 2 * t + 1
                _g(b, rows_b, gsem_b)
                pltpu.make_async_copy(h_sh.at[idx_s.at[a]], rows,
                                      gsem_a).wait()
                _s(rows, a, ssem_a)
                pltpu.make_async_copy(h_sh.at[idx_s.at[b]], rows_b,
                                      gsem_b).wait()
                _s(rows_b, b, ssem_b)
                pltpu.make_async_copy(rows, agg_sh.at[idx_p.at[a]],
                                      ssem_a).wait()

                @pl.when(t < EC // 2 - 1)
                def _():
                    _g(a + 2, rows, gsem_a)
                pltpu.make_async_copy(rows_b, agg_sh.at[idx_p.at[b]],
                                      ssem_b).wait()
                return ()
            lax.fori_loop(0, EC // 2, edge2, ())
            plsc.subcore_barrier()

            pltpu.sync_copy(duidx.at[li, sid], idx_du)
            w_spl = plsc.load_gather(lw_v, [_splat(li)])

            def uchunk(cc, _):
                pltpu.sync_copy(nb.at[idx_du.at[cc]], bias_v)
                pltpu.sync_copy(agg_sh.at[pl.ds(sid * UPT + cc * UCW, UCW)],
                                ubuf)

                def urow(i, _):
                    v = ubuf[i, :]
                    bs = plsc.load_gather(bias_v, [_splat(i)])
                    ubuf[i, :] = _tanh16(v * w_spl + bs)
                    return ()
                lax.fori_loop(0, UCW, urow, ())
                pltpu.sync_copy(ubuf, h_sh.at[idx_du.at[cc]])
                return ()
            lax.fori_loop(0, UC, uchunk, ())
            plsc.subcore_barrier()
            return ()
        lax.fori_loop(0, L, layer, ())

        @pl.when(sid == 0)
        def _():
            pltpu.sync_copy(h_sh.at[ridx_v], rows)

            def mm(r, acc):
                a0, a1 = acc
                v = rows[r, :]
                w0 = plsc.load_gather(wf_v, [_splat(2 * r)])
                w1 = plsc.load_gather(wf_v, [_splat(2 * r + 1)])
                return (a0 + v * w0, a1 + v * w1)
            a0, a1 = lax.fori_loop(
                0, R, mm, (jnp.zeros((NLANE,), jnp.float32),
                           jnp.zeros((NLANE,), jnp.float32)))
            b0 = plsc.load_gather(hb_v, [_splat(0)])
            b1 = plsc.load_gather(hb_v, [_splat(1)])
            ostage[...] = a0 + b0
            pltpu.sync_copy(ostage, out.at[k, 0])
            ostage[...] = a1 + b1
            pltpu.sync_copy(ostage, out.at[k, 1])
        plsc.subcore_barrier()
        return ()
    lax.fori_loop(0, NPASS, one_pass, ())


_sc_call = functools.partial(
    pl.kernel,
    out_type=jax.ShapeDtypeStruct((NPASS * NCORE, C, NLANE), jnp.float32),
    mesh=plsc.VectorSubcoreMesh(core_axis_name="c", subcore_axis_name="s"),
    compiler_params=pltpu.CompilerParams(needs_layout_passes=False,
                                         use_tc_tiling_on_sc=False),
    scratch_types=[
        pltpu.VMEM_SHARED((N_PAD, NLANE), jnp.float32),
        pltpu.VMEM_SHARED((U_PAD, NLANE), jnp.float32),
        pltpu.VMEM((EC, ECW), jnp.int32),
        pltpu.VMEM((EC, ECW), jnp.int32),
        pltpu.VMEM((ECW, NLANE), jnp.float32),
        pltpu.VMEM((ECW, NLANE), jnp.float32),
        pltpu.VMEM((ZROWS, NLANE), jnp.float32),
        pltpu.VMEM((UC, UCW), jnp.int32),
        pltpu.VMEM((UCW,), jnp.float32),
        pltpu.VMEM((UCW, NLANE), jnp.float32),
        pltpu.VMEM((NLANE,), jnp.float32),
        pltpu.VMEM((2 * R,), jnp.float32),
        pltpu.VMEM((NLANE,), jnp.float32),
        pltpu.VMEM((R,), jnp.int32),
        pltpu.VMEM((NLANE,), jnp.float32),
        pltpu.SemaphoreType.DMA,
        pltpu.SemaphoreType.DMA,
        pltpu.SemaphoreType.DMA,
        pltpu.SemaphoreType.DMA,
    ],
)(_sc_body)


def _pad1(a, n, val):
    return jnp.concatenate(
        [a, jnp.full((n - a.shape[0],), val, a.dtype)])


def kernel(X_gene_batch, layer_weight, node_bias, head_W, head_b,
           gene_map, srcs, dst_uniques, dst_poss, root_ids):
    f32, i32 = jnp.float32, jnp.int32

    xt = jnp.zeros((G_PAD, B), f32).at[:G].set(X_gene_batch.T)
    x4 = xt.reshape(G_PAD, NPASS * NCORE, NLANE).transpose(1, 0, 2)

    gidx = _pad1(gene_map.astype(i32), G_PAD, H_GARB).reshape(NSUB, GC, ECW)

    epad = jnp.full((L, E_PAD - E), H_GARB, i32)
    sidx = jnp.concatenate([srcs.astype(i32), epad], axis=1)
    sidx = sidx.reshape(L, NSUB, EC, ECW)
    ppad = jnp.full((L, E_PAD - E), A_GARB, i32)
    pidx = jnp.concatenate([dst_poss.astype(i32), ppad], axis=1)
    pidx = pidx.reshape(L, NSUB, EC, ECW)

    upad = jnp.full((L, U_PAD - U), H_GARB, i32)
    duidx = jnp.concatenate([dst_uniques.astype(i32), upad], axis=1)
    duidx = duidx.reshape(L, NSUB, UC, UCW)
    nb_pad = _pad1(node_bias.astype(f32), N_PAD, 0.0)

    lw16 = _pad1(layer_weight.astype(f32), NLANE, 0.0)
    wf = head_W.astype(f32).reshape(2 * R)
    hb16 = _pad1(head_b.astype(f32), NLANE, 0.0)
    ridx = root_ids.astype(i32)
    zro = jnp.zeros((ZROWS, NLANE), f32)

    out4 = _sc_call(x4, lw16, wf, hb16, gidx, sidx, pidx, duidx, nb_pad,
                    ridx, zro)
    return out4.transpose(0, 2, 1).reshape(B, C)


# speedup vs baseline: 5.3228x; 1.2029x over previous
import functools

import jax
import jax.numpy as jnp
from jax import lax
from jax.experimental import pallas as pl
from jax.experimental.pallas import tpu as pltpu
from jax.experimental.pallas import tpu_sc as plsc

B, G, N, L, E, U, R, C = 64, 20000, 50000, 8, 100000, 25000, 128, 2

NLANE = 16
NCORE = 2
NSUB = 16
NPASS = (B // NLANE) // NCORE

E_PAD = 102400
EC, ECW = 50, 128
U_PAD = 25088
UC, UCW = 14, 112
UPT = UC * UCW
G_PAD = 20480
GC = 10
N_PAD = 50176
ZROWS = 784
H_GARB = N
A_GARB = U


def _splat(i):
    return jnp.full((NLANE,), i, jnp.int32)


def _tanh16(x):
    a = jnp.abs(x)
    e = jnp.exp(-2.0 * a)
    t = (1.0 - e) / (1.0 + e)
    return jnp.where(x < 0, -t, t)


NBUF = 5
ECB = EC // NBUF


def _sc_body(x4, lw16, wf, hb16, gidx, sidx, pidx, duidx, nb, ridx, zro,
             out, h_sh, agg_sh, idx_s, idx_p, rbuf, zbuf, idx_du,
             bias_all, ubuf_a, ubuf_b, featb, lw_v, wf_v, hb_v, ridx_v,
             ostage, gsems, ssems, bsem, zsem):
    cid = lax.axis_index("c")
    sid = lax.axis_index("s")

    pltpu.sync_copy(lw16, lw_v)
    pltpu.sync_copy(wf, wf_v)
    pltpu.sync_copy(hb16, hb_v)
    pltpu.sync_copy(ridx, ridx_v)
    pltpu.sync_copy(zro, zbuf)

    def _zero(dst_sh, base, nblk):
        ds_ = [pltpu.async_copy(
            zbuf, dst_sh.at[pl.ds(base + t * ZROWS, ZROWS)], zsem)
            for t in range(nblk)]
        return ds_

    def do_layer(li, _):
        pltpu.sync_copy(sidx.at[li, sid], idx_s)
        pltpu.sync_copy(pidx.at[li, sid], idx_p)

        def _gw(j, q, do_wait):
            c = (h_sh.at[idx_s.at[j]], rbuf.at[q], gsems[q])
            if do_wait:
                pltpu.make_async_copy(*c).wait()
            else:
                pltpu.async_copy(*c)

        def _sw(j, q, do_wait):
            c = (rbuf.at[q], agg_sh.at[idx_p.at[j]], ssems[q])
            if do_wait:
                pltpu.make_async_copy(*c).wait()
            else:
                pltpu.async_copy(*c, add=True)

        for q in range(NBUF):
            _gw(q, q, False)

        def eblk(t, _):
            base = t * NBUF
            for q in range(NBUF):
                _gw(base + q, q, True)
                _sw(base + q, q, False)
            for q in range(NBUF):
                _sw(base + q, q, True)

                @pl.when(t < ECB - 1)
                def _():
                    _gw(base + NBUF + q, q, False)
            return ()
        lax.fori_loop(0, ECB, eblk, ())
        plsc.subcore_barrier()

        pltpu.sync_copy(duidx.at[li, sid], idx_du)

        def _bias(cc, do_wait):
            c = (nb.at[idx_du.at[cc]],
                 bias_all.at[pl.ds(cc * UCW, UCW)], bsem)
            if do_wait:
                pltpu.make_async_copy(*c).wait()
            else:
                pltpu.async_copy(*c)

        for cc in range(UC):
            _bias(cc, False)
        w_spl = plsc.load_gather(lw_v, [_splat(li)])
        for cc in range(UC):
            _bias(cc, True)

        def _ur(cc, buf, sem, do_wait):
            c = (agg_sh.at[pl.ds(sid * UPT + cc * UCW, UCW)], buf, sem)
            if do_wait:
                pltpu.make_async_copy(*c).wait()
            else:
                pltpu.async_copy(*c)

        def _us(cc, buf, sem, do_wait):
            c = (buf, h_sh.at[idx_du.at[cc]], sem)
            if do_wait:
                pltpu.make_async_copy(*c).wait()
            else:
                pltpu.async_copy(*c)

        def _uz(cc, do_wait):
            c = (zbuf.at[pl.ds(0, UCW)],
                 agg_sh.at[pl.ds(sid * UPT + cc * UCW, UCW)], zsem)
            if do_wait:
                pltpu.make_async_copy(*c).wait()
            else:
                pltpu.async_copy(*c)

        def _compute(buf, cc):
            def _row(i, _):
                v = buf[i, :]
                bs = plsc.load_gather(bias_all, [_splat(cc * UCW + i)])
                buf[i, :] = _tanh16(v * w_spl + bs)
                return ()
            lax.fori_loop(0, UCW, _row, ())

        _ur(0, ubuf_a, gsems[0], False)
        _ur(1, ubuf_b, gsems[1], False)

        def ublk(t, _):
            a = 2 * t
            b = 2 * t + 1
            _ur(a, ubuf_a, gsems[0], True)
            _compute(ubuf_a, a)
            _uz(a, False)
            _us(a, ubuf_a, ssems[0], False)
            _ur(b, ubuf_b, gsems[1], True)
            _compute(ubuf_b, b)
            _uz(b, False)
            _us(b, ubuf_b, ssems[1], False)
            _us(a, ubuf_a, ssems[0], True)

            @pl.when(t < UC // 2 - 1)
            def _():
                _ur(a + 2, ubuf_a, gsems[0], False)
            _us(b, ubuf_b, ssems[1], True)

            @pl.when(t < UC // 2 - 1)
            def _():
                _ur(b + 2, ubuf_b, gsems[1], False)
            return ()
        lax.fori_loop(0, UC // 2, ublk, ())
        for cc in range(UC):
            _uz(cc, True)
        plsc.subcore_barrier()
        return ()

    def one_pass(p, _):
        k = 2 * p + cid

        zds = _zero(h_sh, sid * 4 * ZROWS, 4) + _zero(agg_sh, sid * UPT, 2)
        for d in zds:
            d.wait()
        plsc.subcore_barrier()

        pltpu.sync_copy(gidx.at[sid], idx_s.at[pl.ds(0, GC)])

        def gscat(j, _):
            pltpu.sync_copy(x4.at[k, pl.ds(sid * GC * ECW + j * ECW, ECW)],
                            rbuf.at[0])
            pltpu.sync_copy(rbuf.at[0], h_sh.at[idx_s.at[j]])
            return ()
        lax.fori_loop(0, GC, gscat, ())
        plsc.subcore_barrier()

        lax.fori_loop(0, L, do_layer, ())

        @pl.when(sid == 0)
        def _():
            pltpu.sync_copy(h_sh.at[ridx_v], featb)

            def mm(r, acc):
                a0, a1 = acc
                v = featb[r, :]
                w0 = plsc.load_gather(wf_v, [_splat(2 * r)])
                w1 = plsc.load_gather(wf_v, [_splat(2 * r + 1)])
                return (a0 + v * w0, a1 + v * w1)
            a0, a1 = lax.fori_loop(
                0, R, mm, (jnp.zeros((NLANE,), jnp.float32),
                           jnp.zeros((NLANE,), jnp.float32)))
            b0 = plsc.load_gather(hb_v, [_splat(0)])
            b1 = plsc.load_gather(hb_v, [_splat(1)])
            ostage[...] = a0 + b0
            pltpu.sync_copy(ostage, out.at[k, 0])
            ostage[...] = a1 + b1
            pltpu.sync_copy(ostage, out.at[k, 1])
        plsc.subcore_barrier()
        return ()
    lax.fori_loop(0, NPASS, one_pass, ())


_sc_call = functools.partial(
    pl.kernel,
    out_type=jax.ShapeDtypeStruct((NPASS * NCORE, C, NLANE), jnp.float32),
    mesh=plsc.VectorSubcoreMesh(core_axis_name="c", subcore_axis_name="s"),
    compiler_params=pltpu.CompilerParams(needs_layout_passes=False,
                                         use_tc_tiling_on_sc=False),
    scratch_types=[
        pltpu.VMEM_SHARED((N_PAD, NLANE), jnp.float32),
        pltpu.VMEM_SHARED((U_PAD, NLANE), jnp.float32),
        pltpu.VMEM((EC, ECW), jnp.int32),
        pltpu.VMEM((EC, ECW), jnp.int32),
        pltpu.VMEM((NBUF, ECW, NLANE), jnp.float32),
        pltpu.VMEM((ZROWS, NLANE), jnp.float32),
        pltpu.VMEM((UC, UCW), jnp.int32),
        pltpu.VMEM((UPT,), jnp.float32),
        pltpu.VMEM((UCW, NLANE), jnp.float32),
        pltpu.VMEM((UCW, NLANE), jnp.float32),
        pltpu.VMEM((R, NLANE), jnp.float32),
        pltpu.VMEM((NLANE,), jnp.float32),
        pltpu.VMEM((2 * R,), jnp.float32),
        pltpu.VMEM((NLANE,), jnp.float32),
        pltpu.VMEM((R,), jnp.int32),
        pltpu.VMEM((NLANE,), jnp.float32),
        [pltpu.SemaphoreType.DMA] * NBUF,
        [pltpu.SemaphoreType.DMA] * NBUF,
        pltpu.SemaphoreType.DMA,
        pltpu.SemaphoreType.DMA,
    ],
)(_sc_body)


def _pad1(a, n, val):
    return jnp.concatenate(
        [a, jnp.full((n - a.shape[0],), val, a.dtype)])


def kernel(X_gene_batch, layer_weight, node_bias, head_W, head_b,
           gene_map, srcs, dst_uniques, dst_poss, root_ids):
    f32, i32 = jnp.float32, jnp.int32

    xt = jnp.zeros((G_PAD, B), f32).at[:G].set(X_gene_batch.T)
    x4 = xt.reshape(G_PAD, NPASS * NCORE, NLANE).transpose(1, 0, 2)

    gidx = _pad1(gene_map.astype(i32), G_PAD, H_GARB).reshape(NSUB, GC, ECW)

    epad = jnp.full((L, E_PAD - E), H_GARB, i32)
    sidx = jnp.concatenate([srcs.astype(i32), epad], axis=1)
    sidx = sidx.reshape(L, NSUB, EC, ECW)
    ppad = jnp.full((L, E_PAD - E), A_GARB, i32)
    pidx = jnp.concatenate([dst_poss.astype(i32), ppad], axis=1)
    pidx = pidx.reshape(L, NSUB, EC, ECW)

    upad = jnp.full((L, U_PAD - U), H_GARB, i32)
    duidx = jnp.concatenate([dst_uniques.astype(i32), upad], axis=1)
    duidx = duidx.reshape(L, NSUB, UC, UCW)
    nb_pad = _pad1(node_bias.astype(f32), N_PAD, 0.0)

    lw16 = _pad1(layer_weight.astype(f32), NLANE, 0.0)
    wf = head_W.astype(f32).reshape(2 * R)
    hb16 = _pad1(head_b.astype(f32), NLANE, 0.0)
    ridx = root_ids.astype(i32)
    zro = jnp.zeros((ZROWS, NLANE), f32)

    out4 = _sc_call(x4, lw16, wf, hb16, gidx, sidx, pidx, duidx, nb_pad,
                    ridx, zro)
    return out4.transpose(0, 2, 1).reshape(B, C)


# speedup vs baseline: 5.3967x; 1.0139x over previous
import functools

import jax
import jax.numpy as jnp
from jax import lax
from jax.experimental import pallas as pl
from jax.experimental.pallas import tpu as pltpu
from jax.experimental.pallas import tpu_sc as plsc

B, G, N, L, E, U, R, C = 64, 20000, 50000, 8, 100000, 25000, 128, 2

NLANE = 16
NCORE = 2
NSUB = 16
NPASS = (B // NLANE) // NCORE

E_PAD = 102400
EC, ECW = 25, 256
U_PAD = 25088
UC, UCW = 14, 112
UPT = UC * UCW
G_PAD = 20480
GC = 5
N_PAD = 50176
H_GARB = N
A_GARB = U


def _splat(i):
    return jnp.full((NLANE,), i, jnp.int32)


def _tanh16(x):
    a = jnp.abs(x)
    e = jnp.exp(-2.0 * a)
    t = (1.0 - e) / (1.0 + e)
    return jnp.where(x < 0, -t, t)


NBUF = 5
ECB = EC // NBUF


def _sc_body(x4, lw16, wf, hb16, gidx, sidx, pidx, duidx, nb, ridx, zro,
             out, h_sh, agg_sh, idx_s, idx_p, rbuf, zbuf, idx_du,
             bias_all, ubuf_a, ubuf_b, featb, lw_v, wf_v, hb_v, ridx_v,
             ostage, gsems, ssems, bsem, zsem):
    cid = lax.axis_index("c")
    sid = lax.axis_index("s")

    pltpu.sync_copy(lw16, lw_v)
    pltpu.sync_copy(wf, wf_v)
    pltpu.sync_copy(hb16, hb_v)
    pltpu.sync_copy(ridx, ridx_v)
    pltpu.sync_copy(zro, zbuf)

    def _zero(dst_sh, base, nblk):
        ds_ = [pltpu.async_copy(
            zbuf, dst_sh.at[pl.ds(base + t * UCW, UCW)], zsem)
            for t in range(nblk)]
        return ds_

    def do_layer(li, _):
        pltpu.sync_copy(sidx.at[li, sid], idx_s)
        pltpu.sync_copy(pidx.at[li, sid], idx_p)

        def _gw(j, q, do_wait):
            c = (h_sh.at[idx_s.at[j]], rbuf.at[q], gsems[q])
            if do_wait:
                pltpu.make_async_copy(*c).wait()
            else:
                pltpu.async_copy(*c)

        def _sw(j, q, do_wait):
            c = (rbuf.at[q], agg_sh.at[idx_p.at[j]], ssems[q])
            if do_wait:
                pltpu.make_async_copy(*c).wait()
            else:
                pltpu.async_copy(*c, add=True)

        for q in range(NBUF):
            _gw(q, q, False)

        def eblk(t, _):
            base = t * NBUF
            for q in range(NBUF):
                _gw(base + q, q, True)
                _sw(base + q, q, False)
            for q in range(NBUF):
                _sw(base + q, q, True)

                @pl.when(t < ECB - 1)
                def _():
                    _gw(base + NBUF + q, q, False)
            return ()
        lax.fori_loop(0, ECB, eblk, ())
        plsc.subcore_barrier()

        pltpu.sync_copy(duidx.at[li, sid], idx_du)

        def _bias(cc, do_wait):
            c = (nb.at[idx_du.at[cc]],
                 bias_all.at[pl.ds(cc * UCW, UCW)], bsem)
            if do_wait:
                pltpu.make_async_copy(*c).wait()
            else:
                pltpu.async_copy(*c)

        for cc in range(UC):
            _bias(cc, False)
        w_spl = plsc.load_gather(lw_v, [_splat(li)])
        for cc in range(UC):
            _bias(cc, True)

        def _ur(cc, buf, sem, do_wait):
            c = (agg_sh.at[pl.ds(sid * UPT + cc * UCW, UCW)], buf, sem)
            if do_wait:
                pltpu.make_async_copy(*c).wait()
            else:
                pltpu.async_copy(*c)

        def _us(cc, buf, sem, do_wait):
            c = (buf, h_sh.at[idx_du.at[cc]], sem)
            if do_wait:
                pltpu.make_async_copy(*c).wait()
            else:
                pltpu.async_copy(*c)

        def _uz(cc, do_wait):
            c = (zbuf,
                 agg_sh.at[pl.ds(sid * UPT + cc * UCW, UCW)], zsem)
            if do_wait:
                pltpu.make_async_copy(*c).wait()
            else:
                pltpu.async_copy(*c)

        def _compute(buf, cc):
            def _row(i, _):
                v = buf[i, :]
                bs = plsc.load_gather(bias_all, [_splat(cc * UCW + i)])
                buf[i, :] = _tanh16(v * w_spl + bs)
                return ()
            lax.fori_loop(0, UCW, _row, ())

        _ur(0, ubuf_a, gsems[0], False)
        _ur(1, ubuf_b, gsems[1], False)

        def ublk(t, _):
            a = 2 * t
            b = 2 * t + 1
            _ur(a, ubuf_a, gsems[0], True)
            _compute(ubuf_a, a)
            _uz(a, False)
            _us(a, ubuf_a, ssems[0], False)
            _ur(b, ubuf_b, gsems[1], True)
            _compute(ubuf_b, b)
            _uz(b, False)
            _us(b, ubuf_b, ssems[1], False)
            _us(a, ubuf_a, ssems[0], True)

            @pl.when(t < UC // 2 - 1)
            def _():
                _ur(a + 2, ubuf_a, gsems[0], False)
            _us(b, ubuf_b, ssems[1], True)

            @pl.when(t < UC // 2 - 1)
            def _():
                _ur(b + 2, ubuf_b, gsems[1], False)
            return ()
        lax.fori_loop(0, UC // 2, ublk, ())
        for cc in range(UC):
            _uz(cc, True)
        plsc.subcore_barrier()
        return ()

    def one_pass(p, _):
        k = 2 * p + cid

        zds = _zero(h_sh, sid * 28 * UCW, 28) + _zero(agg_sh, sid * UPT, UC)
        for d in zds:
            d.wait()
        plsc.subcore_barrier()

        pltpu.sync_copy(gidx.at[sid], idx_s.at[pl.ds(0, GC)])

        def gscat(j, _):
            pltpu.sync_copy(x4.at[k, pl.ds(sid * GC * ECW + j * ECW, ECW)],
                            rbuf.at[0])
            pltpu.sync_copy(rbuf.at[0], h_sh.at[idx_s.at[j]])
            return ()
        lax.fori_loop(0, GC, gscat, ())
        plsc.subcore_barrier()

        lax.fori_loop(0, L, do_layer, ())

        @pl.when(sid == 0)
        def _():
            pltpu.sync_copy(h_sh.at[ridx_v], featb)

            def mm(r, acc):
                a0, a1 = acc
                v = featb[r, :]
                w0 = plsc.load_gather(wf_v, [_splat(2 * r)])
                w1 = plsc.load_gather(wf_v, [_splat(2 * r + 1)])
                return (a0 + v * w0, a1 + v * w1)
            a0, a1 = lax.fori_loop(
                0, R, mm, (jnp.zeros((NLANE,), jnp.float32),
                           jnp.zeros((NLANE,), jnp.float32)))
            b0 = plsc.load_gather(hb_v, [_splat(0)])
            b1 = plsc.load_gather(hb_v, [_splat(1)])
            ostage[...] = a0 + b0
            pltpu.sync_copy(ostage, out.at[k, 0])
            ostage[...] = a1 + b1
            pltpu.sync_copy(ostage, out.at[k, 1])
        plsc.subcore_barrier()
        return ()
    lax.fori_loop(0, NPASS, one_pass, ())


_sc_call = functools.partial(
    pl.kernel,
    out_type=jax.ShapeDtypeStruct((NPASS * NCORE, C, NLANE), jnp.float32),
    mesh=plsc.VectorSubcoreMesh(core_axis_name="c", subcore_axis_name="s"),
    compiler_params=pltpu.CompilerParams(needs_layout_passes=False,
                                         use_tc_tiling_on_sc=False),
    scratch_types=[
        pltpu.VMEM_SHARED((N_PAD, NLANE), jnp.float32),
        pltpu.VMEM_SHARED((U_PAD, NLANE), jnp.float32),
        pltpu.VMEM((EC, ECW), jnp.int32),
        pltpu.VMEM((EC, ECW), jnp.int32),
        pltpu.VMEM((NBUF, ECW, NLANE), jnp.float32),
        pltpu.VMEM((UCW, NLANE), jnp.float32),
        pltpu.VMEM((UC, UCW), jnp.int32),
        pltpu.VMEM((UPT,), jnp.float32),
        pltpu.VMEM((UCW, NLANE), jnp.float32),
        pltpu.VMEM((UCW, NLANE), jnp.float32),
        pltpu.VMEM((R, NLANE), jnp.float32),
        pltpu.VMEM((NLANE,), jnp.float32),
        pltpu.VMEM((2 * R,), jnp.float32),
        pltpu.VMEM((NLANE,), jnp.float32),
        pltpu.VMEM((R,), jnp.int32),
        pltpu.VMEM((NLANE,), jnp.float32),
        [pltpu.SemaphoreType.DMA] * NBUF,
        [pltpu.SemaphoreType.DMA] * NBUF,
        pltpu.SemaphoreType.DMA,
        pltpu.SemaphoreType.DMA,
    ],
)(_sc_body)


def _pad1(a, n, val):
    return jnp.concatenate(
        [a, jnp.full((n - a.shape[0],), val, a.dtype)])


def kernel(X_gene_batch, layer_weight, node_bias, head_W, head_b,
           gene_map, srcs, dst_uniques, dst_poss, root_ids):
    f32, i32 = jnp.float32, jnp.int32

    xt = jnp.zeros((G_PAD, B), f32).at[:G].set(X_gene_batch.T)
    x4 = xt.reshape(G_PAD, NPASS * NCORE, NLANE).transpose(1, 0, 2)

    gidx = _pad1(gene_map.astype(i32), G_PAD, H_GARB).reshape(NSUB, GC, ECW)

    epad = jnp.full((L, E_PAD - E), H_GARB, i32)
    sidx = jnp.concatenate([srcs.astype(i32), epad], axis=1)
    sidx = sidx.reshape(L, NSUB, EC, ECW)
    ppad = jnp.full((L, E_PAD - E), A_GARB, i32)
    pidx = jnp.concatenate([dst_poss.astype(i32), ppad], axis=1)
    pidx = pidx.reshape(L, NSUB, EC, ECW)

    upad = jnp.full((L, U_PAD - U), H_GARB, i32)
    duidx = jnp.concatenate([dst_uniques.astype(i32), upad], axis=1)
    duidx = duidx.reshape(L, NSUB, UC, UCW)
    nb_pad = _pad1(node_bias.astype(f32), N_PAD, 0.0)

    lw16 = _pad1(layer_weight.astype(f32), NLANE, 0.0)
    wf = head_W.astype(f32).reshape(2 * R)
    hb16 = _pad1(head_b.astype(f32), NLANE, 0.0)
    ridx = root_ids.astype(i32)
    zro = jnp.zeros((UCW, NLANE), f32)

    out4 = _sc_call(x4, lw16, wf, hb16, gidx, sidx, pidx, duidx, nb_pad,
                    ridx, zro)
    return out4.transpose(0, 2, 1).reshape(B, C)


# speedup vs baseline: 11.0277x; 2.0434x over previous
import functools

import jax
import jax.numpy as jnp
from jax import lax
from jax.experimental import pallas as pl
from jax.experimental.pallas import tpu as pltpu
from jax.experimental.pallas import tpu_sc as plsc

B, G, N, L, E, U, R, C = 64, 20000, 50000, 8, 100000, 25000, 128, 2

NLANE = 16
NCORE = 2
NSUB = 16
NPASS = (B // NLANE) // NCORE

E_PAD = 102400
EC, ECW = 25, 256
U_PAD = 25088
UC, UCW = 14, 112
UPT = UC * UCW
G_PAD = 20480
GC = 5
N_PAD = 50176
H_GARB = N
A_GARB = U


def _splat(i):
    return jnp.full((NLANE,), i, jnp.int32)


def _tanh16(x):
    a = jnp.abs(x)
    e = jnp.exp(-2.0 * a)
    t = (1.0 - e) / (1.0 + e)
    return jnp.where(x < 0, -t, t)


NBUF = 5
ECB = EC // NBUF


def _sc_body(x4, lw16, wf, hb16, gidx, sidx, pidx, duidx, nb, ridx, zro,
             out, h_sh, agg_sh, idx_s, idx_p, rbuf, zbuf, idx_du,
             bias_all, ubuf_a, ubuf_b, featb, lw_v, wf_v, hb_v, ridx_v,
             ostage, gsems, ssems, bsem, zsem):
    cid = lax.axis_index("c")
    sid = lax.axis_index("s")

    pltpu.sync_copy(lw16, lw_v)
    pltpu.sync_copy(wf, wf_v)
    pltpu.sync_copy(hb16, hb_v)
    pltpu.sync_copy(ridx, ridx_v)
    pltpu.sync_copy(zro, zbuf)

    def _zero(dst_sh, base, nblk):
        ds_ = [pltpu.async_copy(
            zbuf, dst_sh.at[pl.ds(base + t * UCW, UCW)], zsem)
            for t in range(nblk)]
        return ds_

    def do_layer(li, _):
        pltpu.sync_copy(sidx.at[li, sid], idx_s)
        pltpu.sync_copy(pidx.at[li, sid], idx_p)

        def _gw(j, q, do_wait):
            c = (h_sh.at[idx_s.at[j]], rbuf.at[q], gsems[q])
            if do_wait:
                pltpu.make_async_copy(*c).wait()
            else:
                pltpu.async_copy(*c)

        def _sw(j, q, do_wait):
            c = (rbuf.at[q], agg_sh.at[idx_p.at[j]], ssems[q])
            if do_wait:
                pltpu.make_async_copy(*c).wait()
            else:
                pltpu.async_copy(*c, add=True)

        for q in range(NBUF):
            _gw(q, q, False)

        def eblk(t, _):
            base = t * NBUF
            for q in range(NBUF):
                _gw(base + q, q, True)
                _sw(base + q, q, False)
            for q in range(NBUF):
                _sw(base + q, q, True)

                @pl.when(t < ECB - 1)
                def _():
                    _gw(base + NBUF + q, q, False)
            return ()
        lax.fori_loop(0, ECB, eblk, ())
        plsc.subcore_barrier()

        pltpu.sync_copy(duidx.at[li, sid], idx_du)

        def _bias(cc, do_wait):
            c = (nb.at[idx_du.at[cc]],
                 bias_all.at[pl.ds(cc * UCW, UCW)], bsem)
            if do_wait:
                pltpu.make_async_copy(*c).wait()
            else:
                pltpu.async_copy(*c)

        for cc in range(UC):
            _bias(cc, False)
        w_spl = plsc.load_gather(lw_v, [_splat(li)])
        for cc in range(UC):
            _bias(cc, True)

        def _ur(cc, buf, sem, do_wait):
            c = (agg_sh.at[pl.ds(sid * UPT + cc * UCW, UCW)], buf, sem)
            if do_wait:
                pltpu.make_async_copy(*c).wait()
            else:
                pltpu.async_copy(*c)

        def _us(cc, buf, sem, do_wait):
            c = (buf, h_sh.at[idx_du.at[cc]], sem)
            if do_wait:
                pltpu.make_async_copy(*c).wait()
            else:
                pltpu.async_copy(*c)

        def _uz(cc, do_wait):
            c = (zbuf,
                 agg_sh.at[pl.ds(sid * UPT + cc * UCW, UCW)], zsem)
            if do_wait:
                pltpu.make_async_copy(*c).wait()
            else:
                pltpu.async_copy(*c)

        def _compute(buf, cc):
            def _grp(g, _):
                bvec = bias_all[pl.ds(cc * UCW + g * NLANE, NLANE)]
                for r in range(NLANE):
                    i = g * NLANE + r
                    bs = bvec.at[jnp.full((NLANE,), r, jnp.int32)].get(
                        mode="promise_in_bounds")
                    buf[i, :] = _tanh16(buf[i, :] * w_spl + bs)
                return ()
            lax.fori_loop(0, UCW // NLANE, _grp, ())

        _ur(0, ubuf_a, gsems[0], False)
        _ur(1, ubuf_b, gsems[1], False)

        def ublk(t, _):
            a = 2 * t
            b = 2 * t + 1
            _ur(a, ubuf_a, gsems[0], True)
            _compute(ubuf_a, a)
            _uz(a, False)
            _us(a, ubuf_a, ssems[0], False)
            _ur(b, ubuf_b, gsems[1], True)
            _compute(ubuf_b, b)
            _uz(b, False)
            _us(b, ubuf_b, ssems[1], False)
            _us(a, ubuf_a, ssems[0], True)

            @pl.when(t < UC // 2 - 1)
            def _():
                _ur(a + 2, ubuf_a, gsems[0], False)
            _us(b, ubuf_b, ssems[1], True)

            @pl.when(t < UC // 2 - 1)
            def _():
                _ur(b + 2, ubuf_b, gsems[1], False)
            return ()
        lax.fori_loop(0, UC // 2, ublk, ())
        for cc in range(UC):
            _uz(cc, True)
        plsc.subcore_barrier()
        return ()

    def one_pass(p, _):
        k = 2 * p + cid

        zds = _zero(h_sh, sid * 28 * UCW, 28) + _zero(agg_sh, sid * UPT, UC)
        for d in zds:
            d.wait()
        plsc.subcore_barrier()

        pltpu.sync_copy(gidx.at[sid], idx_s.at[pl.ds(0, GC)])

        def gscat(j, _):
            pltpu.sync_copy(x4.at[k, pl.ds(sid * GC * ECW + j * ECW, ECW)],
                            rbuf.at[0])
            pltpu.sync_copy(rbuf.at[0], h_sh.at[idx_s.at[j]])
            return ()
        lax.fori_loop(0, GC, gscat, ())
        plsc.subcore_barrier()

        lax.fori_loop(0, L, do_layer, ())

        @pl.when(sid == 0)
        def _():
            pltpu.sync_copy(h_sh.at[ridx_v], featb)

            def mm(r, acc):
                a0, a1 = acc
                v = featb[r, :]
                w0 = plsc.load_gather(wf_v, [_splat(2 * r)])
                w1 = plsc.load_gather(wf_v, [_splat(2 * r + 1)])
                return (a0 + v * w0, a1 + v * w1)
            a0, a1 = lax.fori_loop(
                0, R, mm, (jnp.zeros((NLANE,), jnp.float32),
                           jnp.zeros((NLANE,), jnp.float32)))
            b0 = plsc.load_gather(hb_v, [_splat(0)])
            b1 = plsc.load_gather(hb_v, [_splat(1)])
            ostage[...] = a0 + b0
            pltpu.sync_copy(ostage, out.at[k, 0])
            ostage[...] = a1 + b1
            pltpu.sync_copy(ostage, out.at[k, 1])
        plsc.subcore_barrier()
        return ()
    lax.fori_loop(0, NPASS, one_pass, ())


_sc_call = functools.partial(
    pl.kernel,
    out_type=jax.ShapeDtypeStruct((NPASS * NCORE, C, NLANE), jnp.float32),
    mesh=plsc.VectorSubcoreMesh(core_axis_name="c", subcore_axis_name="s"),
    compiler_params=pltpu.CompilerParams(needs_layout_passes=False,
                                         use_tc_tiling_on_sc=False),
    scratch_types=[
        pltpu.VMEM_SHARED((N_PAD, NLANE), jnp.float32),
        pltpu.VMEM_SHARED((U_PAD, NLANE), jnp.float32),
        pltpu.VMEM((EC, ECW), jnp.int32),
        pltpu.VMEM((EC, ECW), jnp.int32),
        pltpu.VMEM((NBUF, ECW, NLANE), jnp.float32),
        pltpu.VMEM((UCW, NLANE), jnp.float32),
        pltpu.VMEM((UC, UCW), jnp.int32),
        pltpu.VMEM((UPT,), jnp.float32),
        pltpu.VMEM((UCW, NLANE), jnp.float32),
        pltpu.VMEM((UCW, NLANE), jnp.float32),
        pltpu.VMEM((R, NLANE), jnp.float32),
        pltpu.VMEM((NLANE,), jnp.float32),
        pltpu.VMEM((2 * R,), jnp.float32),
        pltpu.VMEM((NLANE,), jnp.float32),
        pltpu.VMEM((R,), jnp.int32),
        pltpu.VMEM((NLANE,), jnp.float32),
        [pltpu.SemaphoreType.DMA] * NBUF,
        [pltpu.SemaphoreType.DMA] * NBUF,
        pltpu.SemaphoreType.DMA,
        pltpu.SemaphoreType.DMA,
    ],
)(_sc_body)


def _pad1(a, n, val):
    return jnp.concatenate(
        [a, jnp.full((n - a.shape[0],), val, a.dtype)])


def kernel(X_gene_batch, layer_weight, node_bias, head_W, head_b,
           gene_map, srcs, dst_uniques, dst_poss, root_ids):
    f32, i32 = jnp.float32, jnp.int32

    xt = jnp.zeros((G_PAD, B), f32).at[:G].set(X_gene_batch.T)
    x4 = xt.reshape(G_PAD, NPASS * NCORE, NLANE).transpose(1, 0, 2)

    gidx = _pad1(gene_map.astype(i32), G_PAD, H_GARB).reshape(NSUB, GC, ECW)

    epad = jnp.full((L, E_PAD - E), H_GARB, i32)
    sidx = jnp.concatenate([srcs.astype(i32), epad], axis=1)
    sidx = sidx.reshape(L, NSUB, EC, ECW)
    ppad = jnp.full((L, E_PAD - E), A_GARB, i32)
    pidx = jnp.concatenate([dst_poss.astype(i32), ppad], axis=1)
    pidx = pidx.reshape(L, NSUB, EC, ECW)

    upad = jnp.full((L, U_PAD - U), H_GARB, i32)
    duidx = jnp.concatenate([dst_uniques.astype(i32), upad], axis=1)
    duidx = duidx.reshape(L, NSUB, UC, UCW)
    nb_pad = _pad1(node_bias.astype(f32), N_PAD, 0.0)

    lw16 = _pad1(layer_weight.astype(f32), NLANE, 0.0)
    wf = head_W.astype(f32).reshape(2 * R)
    hb16 = _pad1(head_b.astype(f32), NLANE, 0.0)
    ridx = root_ids.astype(i32)
    zro = jnp.zeros((UCW, NLANE), f32)

    out4 = _sc_call(x4, lw16, wf, hb16, gidx, sidx, pidx, duidx, nb_pad,
                    ridx, zro)
    return out4.transpose(0, 2, 1).reshape(B, C)


# speedup vs baseline: 11.9355x; 1.0823x over previous
import functools

import jax
import jax.numpy as jnp
from jax import lax
from jax.experimental import pallas as pl
from jax.experimental.pallas import tpu as pltpu
from jax.experimental.pallas import tpu_sc as plsc

B, G, N, L, E, U, R, C = 64, 20000, 50000, 8, 100000, 25000, 128, 2

NLANE = 16
NCORE = 2
NSUB = 16
NPASS = (B // NLANE) // NCORE

E_PAD = 102400
EC, ECW = 25, 256
U_PAD = 25088
UC, UCW = 14, 112
UPT = UC * UCW
G_PAD = 20480
GC = 5
N_PAD = 50176
H_GARB = N
A_GARB = U


def _splat(i):
    return jnp.full((NLANE,), i, jnp.int32)


def _tanh16(x):
    a = jnp.abs(x)
    e = jnp.exp(-2.0 * a)
    t = (1.0 - e) / (1.0 + e)
    return jnp.where(x < 0, -t, t)


NBUF = 5
ECB = EC // NBUF


def _sc_body(x4, lw16, wf, hb16, gidx, sidx, pidx, duidx, nb, ridx, zro,
             out, h_sh, agg_sh, idx_s, idx_p, rbuf, zbuf, idx_du,
             bias_all, ubuf_a, ubuf_b, featb, lw_v, wf_v, hb_v, ridx_v,
             ostage, gsems, ssems, bsem, zsem):
    cid = lax.axis_index("c")
    sid = lax.axis_index("s")

    pltpu.sync_copy(lw16, lw_v)
    pltpu.sync_copy(wf, wf_v)
    pltpu.sync_copy(hb16, hb_v)
    pltpu.sync_copy(ridx, ridx_v)
    pltpu.sync_copy(zro, zbuf)

    def _zero(dst_sh, base, nblk):
        ds_ = [pltpu.async_copy(
            zbuf, dst_sh.at[pl.ds(base + t * UCW, UCW)], zsem)
            for t in range(nblk)]
        return ds_

    def do_layer(li, _):
        pltpu.sync_copy(sidx.at[li, sid], idx_s)
        pltpu.sync_copy(pidx.at[li, sid], idx_p)
        pltpu.sync_copy(duidx.at[li, sid], idx_du)

        def _bias(cc, do_wait):
            c = (nb.at[idx_du.at[cc]],
                 bias_all.at[pl.ds(cc * UCW, UCW)], bsem)
            if do_wait:
                pltpu.make_async_copy(*c).wait()
            else:
                pltpu.async_copy(*c)

        for cc in range(UC):
            _bias(cc, False)

        def _gw(j, q, do_wait):
            c = (h_sh.at[idx_s.at[j]], rbuf.at[q], gsems[q])
            if do_wait:
                pltpu.make_async_copy(*c).wait()
            else:
                pltpu.async_copy(*c)

        def _sw(j, q, do_wait):
            c = (rbuf.at[q], agg_sh.at[idx_p.at[j]], ssems[q])
            if do_wait:
                pltpu.make_async_copy(*c).wait()
            else:
                pltpu.async_copy(*c, add=True)

        for q in range(NBUF):
            _gw(q, q, False)

        def eblk(t, _):
            base = t * NBUF
            for q in range(NBUF):
                _gw(base + q, q, True)
                _sw(base + q, q, False)
            for q in range(NBUF):
                _sw(base + q, q, True)

                @pl.when(t < ECB - 1)
                def _():
                    _gw(base + NBUF + q, q, False)
            return ()
        lax.fori_loop(0, ECB, eblk, ())
        plsc.subcore_barrier()

        w_spl = plsc.load_gather(lw_v, [_splat(li)])
        for cc in range(UC):
            _bias(cc, True)

        def _ur(cc, buf, sem, do_wait):
            c = (agg_sh.at[pl.ds(sid * UPT + cc * UCW, UCW)], buf, sem)
            if do_wait:
                pltpu.make_async_copy(*c).wait()
            else:
                pltpu.async_copy(*c)

        def _us(cc, buf, sem, do_wait):
            c = (buf, h_sh.at[idx_du.at[cc]], sem)
            if do_wait:
                pltpu.make_async_copy(*c).wait()
            else:
                pltpu.async_copy(*c)

        def _uz(cc, do_wait):
            c = (zbuf,
                 agg_sh.at[pl.ds(sid * UPT + cc * UCW, UCW)], zsem)
            if do_wait:
                pltpu.make_async_copy(*c).wait()
            else:
                pltpu.async_copy(*c)

        def _compute(buf, cc):
            def _grp(g, _):
                bvec = bias_all[pl.ds(cc * UCW + g * NLANE, NLANE)]
                for r in range(NLANE):
                    i = g * NLANE + r
                    bs = bvec.at[jnp.full((NLANE,), r, jnp.int32)].get(
                        mode="promise_in_bounds")
                    buf[i, :] = _tanh16(buf[i, :] * w_spl + bs)
                return ()
            lax.fori_loop(0, UCW // NLANE, _grp, ())

        _ur(0, ubuf_a, gsems[0], False)
        _ur(1, ubuf_b, gsems[1], False)

        def ublk(t, _):
            a = 2 * t
            b = 2 * t + 1
            _ur(a, ubuf_a, gsems[0], True)
            _compute(ubuf_a, a)
            _uz(a, False)
            _us(a, ubuf_a, ssems[0], False)
            _ur(b, ubuf_b, gsems[1], True)
            _compute(ubuf_b, b)
            _uz(b, False)
            _us(b, ubuf_b, ssems[1], False)
            _us(a, ubuf_a, ssems[0], True)

            @pl.when(t < UC // 2 - 1)
            def _():
                _ur(a + 2, ubuf_a, gsems[0], False)
            _us(b, ubuf_b, ssems[1], True)

            @pl.when(t < UC // 2 - 1)
            def _():
                _ur(b + 2, ubuf_b, gsems[1], False)
            return ()
        lax.fori_loop(0, UC // 2, ublk, ())
        for cc in range(UC):
            _uz(cc, True)
        plsc.subcore_barrier()
        return ()

    def one_pass(p, _):
        k = 2 * p + cid

        zds = _zero(h_sh, sid * 28 * UCW, 28) + _zero(agg_sh, sid * UPT, UC)
        for d in zds:
            d.wait()
        plsc.subcore_barrier()

        pltpu.sync_copy(gidx.at[sid], idx_s.at[pl.ds(0, GC)])

        def gscat(j, _):
            pltpu.sync_copy(x4.at[k, pl.ds(sid * GC * ECW + j * ECW, ECW)],
                            rbuf.at[0])
            pltpu.sync_copy(rbuf.at[0], h_sh.at[idx_s.at[j]])
            return ()
        lax.fori_loop(0, GC, gscat, ())
        plsc.subcore_barrier()

        lax.fori_loop(0, L, do_layer, ())

        @pl.when(sid == 0)
        def _():
            pltpu.sync_copy(h_sh.at[ridx_v], featb)

            def mm(r, acc):
                a0, a1 = acc
                v = featb[r, :]
                w0 = plsc.load_gather(wf_v, [_splat(2 * r)])
                w1 = plsc.load_gather(wf_v, [_splat(2 * r + 1)])
                return (a0 + v * w0, a1 + v * w1)
            a0, a1 = lax.fori_loop(
                0, R, mm, (jnp.zeros((NLANE,), jnp.float32),
                           jnp.zeros((NLANE,), jnp.float32)))
            b0 = plsc.load_gather(hb_v, [_splat(0)])
            b1 = plsc.load_gather(hb_v, [_splat(1)])
            ostage[...] = a0 + b0
            pltpu.sync_copy(ostage, out.at[k, 0])
            ostage[...] = a1 + b1
            pltpu.sync_copy(ostage, out.at[k, 1])
        plsc.subcore_barrier()
        return ()
    lax.fori_loop(0, NPASS, one_pass, ())


_sc_call = functools.partial(
    pl.kernel,
    out_type=jax.ShapeDtypeStruct((NPASS * NCORE, C, NLANE), jnp.float32),
    mesh=plsc.VectorSubcoreMesh(core_axis_name="c", subcore_axis_name="s"),
    compiler_params=pltpu.CompilerParams(needs_layout_passes=False,
                                         use_tc_tiling_on_sc=False),
    scratch_types=[
        pltpu.VMEM_SHARED((N_PAD, NLANE), jnp.float32),
        pltpu.VMEM_SHARED((U_PAD, NLANE), jnp.float32),
        pltpu.VMEM((EC, ECW), jnp.int32),
        pltpu.VMEM((EC, ECW), jnp.int32),
        pltpu.VMEM((NBUF, ECW, NLANE), jnp.float32),
        pltpu.VMEM((UCW, NLANE), jnp.float32),
        pltpu.VMEM((UC, UCW), jnp.int32),
        pltpu.VMEM((UPT,), jnp.float32),
        pltpu.VMEM((UCW, NLANE), jnp.float32),
        pltpu.VMEM((UCW, NLANE), jnp.float32),
        pltpu.VMEM((R, NLANE), jnp.float32),
        pltpu.VMEM((NLANE,), jnp.float32),
        pltpu.VMEM((2 * R,), jnp.float32),
        pltpu.VMEM((NLANE,), jnp.float32),
        pltpu.VMEM((R,), jnp.int32),
        pltpu.VMEM((NLANE,), jnp.float32),
        [pltpu.SemaphoreType.DMA] * NBUF,
        [pltpu.SemaphoreType.DMA] * NBUF,
        pltpu.SemaphoreType.DMA,
        pltpu.SemaphoreType.DMA,
    ],
)(_sc_body)


def _pad1(a, n, val):
    return jnp.concatenate(
        [a, jnp.full((n - a.shape[0],), val, a.dtype)])


def kernel(X_gene_batch, layer_weight, node_bias, head_W, head_b,
           gene_map, srcs, dst_uniques, dst_poss, root_ids):
    f32, i32 = jnp.float32, jnp.int32

    xt = jnp.zeros((G_PAD, B), f32).at[:G].set(X_gene_batch.T)
    x4 = xt.reshape(G_PAD, NPASS * NCORE, NLANE).transpose(1, 0, 2)

    gidx = _pad1(gene_map.astype(i32), G_PAD, H_GARB).reshape(NSUB, GC, ECW)

    epad = jnp.full((L, E_PAD - E), H_GARB, i32)
    sidx = jnp.concatenate([srcs.astype(i32), epad], axis=1)
    sidx = sidx.reshape(L, NSUB, EC, ECW)
    ppad = jnp.full((L, E_PAD - E), A_GARB, i32)
    pidx = jnp.concatenate([dst_poss.astype(i32), ppad], axis=1)
    pidx = pidx.reshape(L, NSUB, EC, ECW)

    upad = jnp.full((L, U_PAD - U), H_GARB, i32)
    duidx = jnp.concatenate([dst_uniques.astype(i32), upad], axis=1)
    duidx = duidx.reshape(L, NSUB, UC, UCW)
    nb_pad = _pad1(node_bias.astype(f32), N_PAD, 0.0)

    lw16 = _pad1(layer_weight.astype(f32), NLANE, 0.0)
    wf = head_W.astype(f32).reshape(2 * R)
    hb16 = _pad1(head_b.astype(f32), NLANE, 0.0)
    ridx = root_ids.astype(i32)
    zro = jnp.zeros((UCW, NLANE), f32)

    out4 = _sc_call(x4, lw16, wf, hb16, gidx, sidx, pidx, duidx, nb_pad,
                    ridx, zro)
    return out4.transpose(0, 2, 1).reshape(B, C)


# speedup vs baseline: 11.9752x; 1.0033x over previous
import functools

import jax
import jax.numpy as jnp
from jax import lax
from jax.experimental import pallas as pl
from jax.experimental.pallas import tpu as pltpu
from jax.experimental.pallas import tpu_sc as plsc

B, G, N, L, E, U, R, C = 64, 20000, 50000, 8, 100000, 25000, 128, 2

NLANE = 16
NCORE = 2
NSUB = 16
NPASS = (B // NLANE) // NCORE

E_PAD = 102400
EC, ECW = 20, 320
U_PAD = 25088
UC, UCW = 14, 112
UPT = UC * UCW
G_PAD = 20480
GC = 4
N_PAD = 50176
H_GARB = N
A_GARB = U


def _splat(i):
    return jnp.full((NLANE,), i, jnp.int32)


def _tanh16(x):
    a = jnp.abs(x)
    e = jnp.exp(-2.0 * a)
    t = (1.0 - e) / (1.0 + e)
    return jnp.where(x < 0, -t, t)


NBUF = 4
ECB = EC // NBUF


def _sc_body(x4, lw16, wf, hb16, gidx, sidx, pidx, duidx, nb, ridx, zro,
             out, h_sh, agg_sh, idx_s, idx_p, rbuf, zbuf, idx_du,
             bias_all, ubuf_a, ubuf_b, featb, lw_v, wf_v, hb_v, ridx_v,
             ostage, gsems, ssems, bsem, zsem):
    cid = lax.axis_index("c")
    sid = lax.axis_index("s")

    pltpu.sync_copy(lw16, lw_v)
    pltpu.sync_copy(wf, wf_v)
    pltpu.sync_copy(hb16, hb_v)
    pltpu.sync_copy(ridx, ridx_v)
    pltpu.sync_copy(zro, zbuf)

    def _zero(dst_sh, base, nblk):
        ds_ = [pltpu.async_copy(
            zbuf, dst_sh.at[pl.ds(base + t * UCW, UCW)], zsem)
            for t in range(nblk)]
        return ds_

    def do_layer(li, _):
        pltpu.sync_copy(sidx.at[li, sid], idx_s)
        pltpu.sync_copy(pidx.at[li, sid], idx_p)
        pltpu.sync_copy(duidx.at[li, sid], idx_du)

        def _bias(cc, do_wait):
            c = (nb.at[idx_du.at[cc]],
                 bias_all.at[pl.ds(cc * UCW, UCW)], bsem)
            if do_wait:
                pltpu.make_async_copy(*c).wait()
            else:
                pltpu.async_copy(*c)

        for cc in range(UC):
            _bias(cc, False)

        def _gw(j, q, do_wait):
            c = (h_sh.at[idx_s.at[j]], rbuf.at[q], gsems[q])
            if do_wait:
                pltpu.make_async_copy(*c).wait()
            else:
                pltpu.async_copy(*c)

        def _sw(j, q, do_wait):
            c = (rbuf.at[q], agg_sh.at[idx_p.at[j]], ssems[q])
            if do_wait:
                pltpu.make_async_copy(*c).wait()
            else:
                pltpu.async_copy(*c, add=True)

        for q in range(NBUF):
            _gw(q, q, False)

        def eblk(t, _):
            base = t * NBUF
            for q in range(NBUF):
                _gw(base + q, q, True)
                _sw(base + q, q, False)
            for q in range(NBUF):
                _sw(base + q, q, True)

                @pl.when(t < ECB - 1)
                def _():
                    _gw(base + NBUF + q, q, False)
            return ()
        lax.fori_loop(0, ECB, eblk, ())
        plsc.subcore_barrier()

        w_spl = plsc.load_gather(lw_v, [_splat(li)])
        for cc in range(UC):
            _bias(cc, True)

        def _ur(cc, buf, sem, do_wait):
            c = (agg_sh.at[pl.ds(sid * UPT + cc * UCW, UCW)], buf, sem)
            if do_wait:
                pltpu.make_async_copy(*c).wait()
            else:
                pltpu.async_copy(*c)

        def _us(cc, buf, sem, do_wait):
            c = (buf, h_sh.at[idx_du.at[cc]], sem)
            if do_wait:
                pltpu.make_async_copy(*c).wait()
            else:
                pltpu.async_copy(*c)

        def _uz(cc, do_wait):
            c = (zbuf,
                 agg_sh.at[pl.ds(sid * UPT + cc * UCW, UCW)], zsem)
            if do_wait:
                pltpu.make_async_copy(*c).wait()
            else:
                pltpu.async_copy(*c)

        def _compute(buf, cc):
            def _grp(g, _):
                bvec = bias_all[pl.ds(cc * UCW + g * NLANE, NLANE)]
                for r in range(NLANE):
                    i = g * NLANE + r
                    bs = bvec.at[jnp.full((NLANE,), r, jnp.int32)].get(
                        mode="promise_in_bounds")
                    buf[i, :] = _tanh16(buf[i, :] * w_spl + bs)
                return ()
            lax.fori_loop(0, UCW // NLANE, _grp, ())

        _ur(0, ubuf_a, gsems[0], False)
        _ur(1, ubuf_b, gsems[1], False)

        def ublk(t, _):
            a = 2 * t
            b = 2 * t + 1
            _ur(a, ubuf_a, gsems[0], True)
            _compute(ubuf_a, a)
            _uz(a, False)
            _us(a, ubuf_a, ssems[0], False)
            _ur(b, ubuf_b, gsems[1], True)
            _compute(ubuf_b, b)
            _uz(b, False)
            _us(b, ubuf_b, ssems[1], False)
            _us(a, ubuf_a, ssems[0], True)

            @pl.when(t < UC // 2 - 1)
            def _():
                _ur(a + 2, ubuf_a, gsems[0], False)
            _us(b, ubuf_b, ssems[1], True)

            @pl.when(t < UC // 2 - 1)
            def _():
                _ur(b + 2, ubuf_b, gsems[1], False)
            return ()
        lax.fori_loop(0, UC // 2, ublk, ())
        for cc in range(UC):
            _uz(cc, True)
        plsc.subcore_barrier()
        return ()

    def one_pass(p, _):
        k = 2 * p + cid

        zds = _zero(h_sh, sid * 28 * UCW, 28) + _zero(agg_sh, sid * UPT, UC)
        for d in zds:
            d.wait()
        plsc.subcore_barrier()

        pltpu.sync_copy(gidx.at[sid], idx_s.at[pl.ds(0, GC)])

        def gscat(j, _):
            pltpu.sync_copy(x4.at[k, pl.ds(sid * GC * ECW + j * ECW, ECW)],
                            rbuf.at[0])
            pltpu.sync_copy(rbuf.at[0], h_sh.at[idx_s.at[j]])
            return ()
        lax.fori_loop(0, GC, gscat, ())
        plsc.subcore_barrier()

        lax.fori_loop(0, L, do_layer, ())

        @pl.when(sid == 0)
        def _():
            pltpu.sync_copy(h_sh.at[ridx_v], featb)

            def mm(r, acc):
                a0, a1 = acc
                v = featb[r, :]
                w0 = plsc.load_gather(wf_v, [_splat(2 * r)])
                w1 = plsc.load_gather(wf_v, [_splat(2 * r + 1)])
                return (a0 + v * w0, a1 + v * w1)
            a0, a1 = lax.fori_loop(
                0, R, mm, (jnp.zeros((NLANE,), jnp.float32),
                           jnp.zeros((NLANE,), jnp.float32)))
            b0 = plsc.load_gather(hb_v, [_splat(0)])
            b1 = plsc.load_gather(hb_v, [_splat(1)])
            ostage[...] = a0 + b0
            pltpu.sync_copy(ostage, out.at[k, 0])
            ostage[...] = a1 + b1
            pltpu.sync_copy(ostage, out.at[k, 1])
        plsc.subcore_barrier()
        return ()
    lax.fori_loop(0, NPASS, one_pass, ())


_sc_call = functools.partial(
    pl.kernel,
    out_type=jax.ShapeDtypeStruct((NPASS * NCORE, C, NLANE), jnp.float32),
    mesh=plsc.VectorSubcoreMesh(core_axis_name="c", subcore_axis_name="s"),
    compiler_params=pltpu.CompilerParams(needs_layout_passes=False,
                                         use_tc_tiling_on_sc=False),
    scratch_types=[
        pltpu.VMEM_SHARED((N_PAD, NLANE), jnp.float32),
        pltpu.VMEM_SHARED((U_PAD, NLANE), jnp.float32),
        pltpu.VMEM((EC, ECW), jnp.int32),
        pltpu.VMEM((EC, ECW), jnp.int32),
        pltpu.VMEM((NBUF, ECW, NLANE), jnp.float32),
        pltpu.VMEM((UCW, NLANE), jnp.float32),
        pltpu.VMEM((UC, UCW), jnp.int32),
        pltpu.VMEM((UPT,), jnp.float32),
        pltpu.VMEM((UCW, NLANE), jnp.float32),
        pltpu.VMEM((UCW, NLANE), jnp.float32),
        pltpu.VMEM((R, NLANE), jnp.float32),
        pltpu.VMEM((NLANE,), jnp.float32),
        pltpu.VMEM((2 * R,), jnp.float32),
        pltpu.VMEM((NLANE,), jnp.float32),
        pltpu.VMEM((R,), jnp.int32),
        pltpu.VMEM((NLANE,), jnp.float32),
        [pltpu.SemaphoreType.DMA] * NBUF,
        [pltpu.SemaphoreType.DMA] * NBUF,
        pltpu.SemaphoreType.DMA,
        pltpu.SemaphoreType.DMA,
    ],
)(_sc_body)


def _pad1(a, n, val):
    return jnp.concatenate(
        [a, jnp.full((n - a.shape[0],), val, a.dtype)])


def kernel(X_gene_batch, layer_weight, node_bias, head_W, head_b,
           gene_map, srcs, dst_uniques, dst_poss, root_ids):
    f32, i32 = jnp.float32, jnp.int32

    xt = jnp.zeros((G_PAD, B), f32).at[:G].set(X_gene_batch.T)
    x4 = xt.reshape(G_PAD, NPASS * NCORE, NLANE).transpose(1, 0, 2)

    gidx = _pad1(gene_map.astype(i32), G_PAD, H_GARB).reshape(NSUB, GC, ECW)

    epad = jnp.full((L, E_PAD - E), H_GARB, i32)
    sidx = jnp.concatenate([srcs.astype(i32), epad], axis=1)
    sidx = sidx.reshape(L, NSUB, EC, ECW)
    ppad = jnp.full((L, E_PAD - E), A_GARB, i32)
    pidx = jnp.concatenate([dst_poss.astype(i32), ppad], axis=1)
    pidx = pidx.reshape(L, NSUB, EC, ECW)

    upad = jnp.full((L, U_PAD - U), H_GARB, i32)
    duidx = jnp.concatenate([dst_uniques.astype(i32), upad], axis=1)
    duidx = duidx.reshape(L, NSUB, UC, UCW)
    nb_pad = _pad1(node_bias.astype(f32), N_PAD, 0.0)

    lw16 = _pad1(layer_weight.astype(f32), NLANE, 0.0)
    wf = head_W.astype(f32).reshape(2 * R)
    hb16 = _pad1(head_b.astype(f32), NLANE, 0.0)
    ridx = root_ids.astype(i32)
    zro = jnp.zeros((UCW, NLANE), f32)

    out4 = _sc_call(x4, lw16, wf, hb16, gidx, sidx, pidx, duidx, nb_pad,
                    ridx, zro)
    return out4.transpose(0, 2, 1).reshape(B, C)


# speedup vs baseline: 12.1065x; 1.0110x over previous
import functools

import jax
import jax.numpy as jnp
from jax import lax
from jax.experimental import pallas as pl
from jax.experimental.pallas import tpu as pltpu
from jax.experimental.pallas import tpu_sc as plsc

B, G, N, L, E, U, R, C = 64, 20000, 50000, 8, 100000, 25000, 128, 2

NLANE = 16
NCORE = 2
NSUB = 16
NPASS = (B // NLANE) // NCORE

E_PAD = 102400
EC, ECW = 20, 320
U_PAD = 25088
UC, UCW = 7, 224
UPT = UC * UCW
NZH = 3136 // UCW
G_PAD = 20480
GC = 4
N_PAD = 50176
H_GARB = N
A_GARB = U


def _splat(i):
    return jnp.full((NLANE,), i, jnp.int32)


def _tanh16(x):
    a = jnp.abs(x)
    e = jnp.exp(-2.0 * a)
    t = (1.0 - e) / (1.0 + e)
    return jnp.where(x < 0, -t, t)


NBUF = 4
ECB = EC // NBUF


def _sc_body(x4, lw16, wf, hb16, gidx, sidx, pidx, duidx, nb, ridx, zro,
             out, h_sh, agg_sh, idx_s, idx_p, rbuf, zbuf, idx_du,
             bias_all, ubuf_a, ubuf_b, featb, lw_v, wf_v, hb_v, ridx_v,
             ostage, gsems, ssems, bsem, zsem):
    cid = lax.axis_index("c")
    sid = lax.axis_index("s")

    pltpu.sync_copy(lw16, lw_v)
    pltpu.sync_copy(wf, wf_v)
    pltpu.sync_copy(hb16, hb_v)
    pltpu.sync_copy(ridx, ridx_v)
    pltpu.sync_copy(zro, zbuf)

    def _zero(dst_sh, base, nblk):
        ds_ = [pltpu.async_copy(
            zbuf, dst_sh.at[pl.ds(base + t * UCW, UCW)], zsem)
            for t in range(nblk)]
        return ds_

    def do_layer(li, _):
        pltpu.sync_copy(sidx.at[li, sid], idx_s)
        pltpu.sync_copy(pidx.at[li, sid], idx_p)
        pltpu.sync_copy(duidx.at[li, sid], idx_du)

        def _bias(cc, do_wait):
            c = (nb.at[idx_du.at[cc]],
                 bias_all.at[pl.ds(cc * UCW, UCW)], bsem)
            if do_wait:
                pltpu.make_async_copy(*c).wait()
            else:
                pltpu.async_copy(*c)

        for cc in range(UC):
            _bias(cc, False)

        def _gw(j, q, do_wait):
            c = (h_sh.at[idx_s.at[j]], rbuf.at[q], gsems[q])
            if do_wait:
                pltpu.make_async_copy(*c).wait()
            else:
                pltpu.async_copy(*c)

        def _sw(j, q, do_wait):
            c = (rbuf.at[q], agg_sh.at[idx_p.at[j]], ssems[q])
            if do_wait:
                pltpu.make_async_copy(*c).wait()
            else:
                pltpu.async_copy(*c, add=True)

        for q in range(NBUF):
            _gw(q, q, False)

        def eblk(t, _):
            base = t * NBUF
            for q in range(NBUF):
                _gw(base + q, q, True)
                _sw(base + q, q, False)
            for q in range(NBUF):
                _sw(base + q, q, True)

                @pl.when(t < ECB - 1)
                def _():
                    _gw(base + NBUF + q, q, False)
            return ()
        lax.fori_loop(0, ECB, eblk, ())
        plsc.subcore_barrier()

        w_spl = plsc.load_gather(lw_v, [_splat(li)])
        for cc in range(UC):
            _bias(cc, True)

        def _ur(cc, buf, sem, do_wait):
            c = (agg_sh.at[pl.ds(sid * UPT + cc * UCW, UCW)], buf, sem)
            if do_wait:
                pltpu.make_async_copy(*c).wait()
            else:
                pltpu.async_copy(*c)

        def _us(cc, buf, sem, do_wait):
            c = (buf, h_sh.at[idx_du.at[cc]], sem)
            if do_wait:
                pltpu.make_async_copy(*c).wait()
            else:
                pltpu.async_copy(*c)

        def _uz(cc, do_wait):
            c = (zbuf,
                 agg_sh.at[pl.ds(sid * UPT + cc * UCW, UCW)], zsem)
            if do_wait:
                pltpu.make_async_copy(*c).wait()
            else:
                pltpu.async_copy(*c)

        def _compute(buf, cc):
            def _grp(g, _):
                bvec = bias_all[pl.ds(cc * UCW + g * NLANE, NLANE)]
                for r in range(NLANE):
                    i = g * NLANE + r
                    bs = bvec.at[jnp.full((NLANE,), r, jnp.int32)].get(
                        mode="promise_in_bounds")
                    buf[i, :] = _tanh16(buf[i, :] * w_spl + bs)
                return ()
            lax.fori_loop(0, UCW // NLANE, _grp, ())

        _ur(0, ubuf_a, gsems[0], False)
        _ur(0, ubuf_a, gsems[0], True)
        _compute(ubuf_a, 0)
        _uz(0, False)
        _us(0, ubuf_a, ssems[0], False)
        _us(0, ubuf_a, ssems[0], True)
        _ur(1, ubuf_a, gsems[0], False)
        _ur(2, ubuf_b, gsems[1], False)

        def ublk(t, _):
            a = 2 * t + 1
            b = 2 * t + 2
            _ur(a, ubuf_a, gsems[0], True)
            _compute(ubuf_a, a)
            _uz(a, False)
            _us(a, ubuf_a, ssems[0], False)
            _ur(b, ubuf_b, gsems[1], True)
            _compute(ubuf_b, b)
            _uz(b, False)
            _us(b, ubuf_b, ssems[1], False)
            _us(a, ubuf_a, ssems[0], True)

            @pl.when(t < UC // 2 - 1)
            def _():
                _ur(a + 2, ubuf_a, gsems[0], False)
            _us(b, ubuf_b, ssems[1], True)

            @pl.when(t < UC // 2 - 1)
            def _():
                _ur(b + 2, ubuf_b, gsems[1], False)
            return ()
        lax.fori_loop(0, UC // 2, ublk, ())
        for cc in range(UC):
            _uz(cc, True)
        plsc.subcore_barrier()
        return ()

    def one_pass(p, _):
        k = 2 * p + cid

        zds = _zero(h_sh, sid * NZH * UCW, NZH) + _zero(agg_sh, sid * UPT, UC)
        for d in zds:
            d.wait()
        plsc.subcore_barrier()

        pltpu.sync_copy(gidx.at[sid], idx_s.at[pl.ds(0, GC)])
        gx = [pltpu.async_copy(
            x4.at[k, pl.ds(sid * GC * ECW + j * ECW, ECW)],
            rbuf.at[j], gsems[j]) for j in range(GC)]
        gs = []
        for j in range(GC):
            gx[j].wait()
            gs.append(pltpu.async_copy(rbuf.at[j], h_sh.at[idx_s.at[j]],
                                       ssems[j]))
        for d in gs:
            d.wait()
        plsc.subcore_barrier()

        lax.fori_loop(0, L, do_layer, ())

        @pl.when(sid == 0)
        def _():
            pltpu.sync_copy(h_sh.at[ridx_v], featb)

            def mm(r, acc):
                a0, a1 = acc
                v = featb[r, :]
                w0 = plsc.load_gather(wf_v, [_splat(2 * r)])
                w1 = plsc.load_gather(wf_v, [_splat(2 * r + 1)])
                return (a0 + v * w0, a1 + v * w1)
            a0, a1 = lax.fori_loop(
                0, R, mm, (jnp.zeros((NLANE,), jnp.float32),
                           jnp.zeros((NLANE,), jnp.float32)))
            b0 = plsc.load_gather(hb_v, [_splat(0)])
            b1 = plsc.load_gather(hb_v, [_splat(1)])
            ostage[...] = a0 + b0
            pltpu.sync_copy(ostage, out.at[k, 0])
            ostage[...] = a1 + b1
            pltpu.sync_copy(ostage, out.at[k, 1])
        plsc.subcore_barrier()
        return ()
    lax.fori_loop(0, NPASS, one_pass, ())


_sc_call = functools.partial(
    pl.kernel,
    out_type=jax.ShapeDtypeStruct((NPASS * NCORE, C, NLANE), jnp.float32),
    mesh=plsc.VectorSubcoreMesh(core_axis_name="c", subcore_axis_name="s"),
    compiler_params=pltpu.CompilerParams(needs_layout_passes=False,
                                         use_tc_tiling_on_sc=False),
    scratch_types=[
        pltpu.VMEM_SHARED((N_PAD, NLANE), jnp.float32),
        pltpu.VMEM_SHARED((U_PAD, NLANE), jnp.float32),
        pltpu.VMEM((EC, ECW), jnp.int32),
        pltpu.VMEM((EC, ECW), jnp.int32),
        pltpu.VMEM((NBUF, ECW, NLANE), jnp.float32),
        pltpu.VMEM((UCW, NLANE), jnp.float32),
        pltpu.VMEM((UC, UCW), jnp.int32),
        pltpu.VMEM((UPT,), jnp.float32),
        pltpu.VMEM((UCW, NLANE), jnp.float32),
        pltpu.VMEM((UCW, NLANE), jnp.float32),
        pltpu.VMEM((R, NLANE), jnp.float32),
        pltpu.VMEM((NLANE,), jnp.float32),
        pltpu.VMEM((2 * R,), jnp.float32),
        pltpu.VMEM((NLANE,), jnp.float32),
        pltpu.VMEM((R,), jnp.int32),
        pltpu.VMEM((NLANE,), jnp.float32),
        [pltpu.SemaphoreType.DMA] * NBUF,
        [pltpu.SemaphoreType.DMA] * NBUF,
        pltpu.SemaphoreType.DMA,
        pltpu.SemaphoreType.DMA,
    ],
)(_sc_body)


def _pad1(a, n, val):
    return jnp.concatenate(
        [a, jnp.full((n - a.shape[0],), val, a.dtype)])


def kernel(X_gene_batch, layer_weight, node_bias, head_W, head_b,
           gene_map, srcs, dst_uniques, dst_poss, root_ids):
    f32, i32 = jnp.float32, jnp.int32

    xt = jnp.zeros((G_PAD, B), f32).at[:G].set(X_gene_batch.T)
    x4 = xt.reshape(G_PAD, NPASS * NCORE, NLANE).transpose(1, 0, 2)

    gidx = _pad1(gene_map.astype(i32), G_PAD, H_GARB).reshape(NSUB, GC, ECW)

    epad = jnp.full((L, E_PAD - E), H_GARB, i32)
    sidx = jnp.concatenate([srcs.astype(i32), epad], axis=1)
    sidx = sidx.reshape(L, NSUB, EC, ECW)
    ppad = jnp.full((L, E_PAD - E), A_GARB, i32)
    pidx = jnp.concatenate([dst_poss.astype(i32), ppad], axis=1)
    pidx = pidx.reshape(L, NSUB, EC, ECW)

    upad = jnp.full((L, U_PAD - U), H_GARB, i32)
    duidx = jnp.concatenate([dst_uniques.astype(i32), upad], axis=1)
    duidx = duidx.reshape(L, NSUB, UC, UCW)
    nb_pad = _pad1(node_bias.astype(f32), N_PAD, 0.0)

    lw16 = _pad1(layer_weight.astype(f32), NLANE, 0.0)
    wf = head_W.astype(f32).reshape(2 * R)
    hb16 = _pad1(head_b.astype(f32), NLANE, 0.0)
    ridx = root_ids.astype(i32)
    zro = jnp.zeros((UCW, NLANE), f32)

    out4 = _sc_call(x4, lw16, wf, hb16, gidx, sidx, pidx, duidx, nb_pad,
                    ridx, zro)
    return out4.transpose(0, 2, 1).reshape(B, C)
